# Initial kernel scaffold; baseline (speedup 1.0000x reference)
import jax
import jax.numpy as jnp
from jax.experimental import pallas as pl


def kernel(x, edge_index_r0, edge_index_r1, W1_r0, b1_r0, W1_r1, b1_r1, W6_r0, b6_r0, W6_r1, b6_r1):
    raise NotImplementedError("write your pallas kernel here")


# speedup vs baseline: 2.1919x; 2.1919x over previous
import functools

import jax
import jax.numpy as jnp
from jax import lax
from jax.experimental import pallas as pl
from jax.experimental.pallas import tpu as pltpu
from jax.experimental.pallas import tpu_sc as plsc

N = 10000
D = 128
E = 320000
NCORE = 2
NSUB = 16
NW = NCORE * NSUB
BK = 128
NB = 80
EPW = NB * BK
EPAD = NW * EPW
NPAD = 10240
RPS = NPAD // NSUB
PADIDX = N

_mesh = plsc.VectorSubcoreMesh(
    core_axis_name="c", subcore_axis_name="s",
    num_cores=NCORE, num_subcores=NSUB,
)


CH = 16
NCH = NB // CH
HB = NPAD // BK
KD = 2048
NCHK = EPAD // KD


def _deg_body(idx_ref, out_ref):
    g = pl.program_id(1)
    idx = idx_ref[0, 0, :]
    hi = lax.shift_right_logical(idx, 7)
    lo = lax.bitwise_and(idx, 127)
    rh = lax.broadcasted_iota(jnp.int32, (KD, HB), 1)
    rl = lax.broadcasted_iota(jnp.int32, (KD, BK), 1)
    ohh = (hi[:, None] == rh).astype(jnp.bfloat16)
    ohl = (lo[:, None] == rl).astype(jnp.bfloat16)
    part = lax.dot_general(ohh, ohl, (((0,), (0,)), ((), ())),
                           preferred_element_type=jnp.float32)

    @pl.when(g == 0)
    def _():
        out_ref[...] = jnp.zeros_like(out_ref)

    out_ref[...] += part[None]


_deg_tc = pl.pallas_call(
    _deg_body,
    grid=(4, NCHK),
    in_specs=[pl.BlockSpec((1, 1, KD), lambda a, g: (a, 0, g))],
    out_specs=pl.BlockSpec((1, HB, BK), lambda a, g: (a, 0, 0)),
    out_shape=jax.ShapeDtypeStruct((4, HB, BK), jnp.float32),
)


@functools.partial(
    pl.kernel,
    out_type=jax.ShapeDtypeStruct((2, NCORE, NPAD, D), jnp.float32),
    mesh=_mesh,
    scratch_types=[
        pltpu.VMEM((CH, BK), jnp.int32),
        pltpu.VMEM((CH, BK), jnp.int32),
        pltpu.VMEM((BK, D), jnp.float32),
        pltpu.VMEM((BK, D), jnp.float32),
        pltpu.VMEM_SHARED((NPAD, D), jnp.float32),
        pltpu.SemaphoreType.DMA,
        pltpu.SemaphoreType.DMA,
    ],
)
def _spmv_kernel(t0, s0, d0, t1, s1, d1, zeros_hbm, out,
                 src_v, dst_v, msg0, msg1, acc, sem0, sem1):
    c = lax.axis_index("c")
    s = lax.axis_index("s")
    w = c * NSUB + s
    for rel, (th, sh, dh) in enumerate(((t0, s0, d0), (t1, s1, d1))):
        pltpu.sync_copy(zeros_hbm, acc.at[pl.ds(s * RPS, RPS)])
        plsc.subcore_barrier()

        def chunk(cc, _, th=th, sh=sh, dh=dh):
            pltpu.sync_copy(sh.at[w, pl.ds(cc * CH, CH)], src_v)
            pltpu.sync_copy(dh.at[w, pl.ds(cc * CH, CH)], dst_v)
            for j0 in range(0, CH, 2):
                j1 = j0 + 1
                g0 = pltpu.async_copy(th.at[src_v.at[j0]], msg0, sem0)
                g1 = pltpu.async_copy(th.at[src_v.at[j1]], msg1, sem1)
                g0.wait()
                pltpu.sync_copy(msg0, acc.at[dst_v.at[j0]], add=True)
                g1.wait()
                pltpu.sync_copy(msg1, acc.at[dst_v.at[j1]], add=True)
            return 0

        lax.fori_loop(0, NCH, chunk, 0)
        plsc.subcore_barrier()
        pltpu.sync_copy(acc.at[pl.ds(s * RPS, RPS)],
                        out.at[rel, c, pl.ds(s * RPS, RPS)])
        plsc.subcore_barrier()


BN = 1000


def _prep_body(x_ref, degp_ref, xn0_ref, xn1_ref):
    dg0 = degp_ref[:, 0]
    dg1 = degp_ref[:, 2]
    ns0 = lax.rsqrt(jnp.maximum(dg0, 1.0))
    ns1 = lax.rsqrt(jnp.maximum(dg1, 1.0))
    xv = x_ref[...]
    xn0_ref[...] = xv * ns0[:, None]
    xn1_ref[...] = xv * ns1[:, None]


_prep = pl.pallas_call(
    _prep_body,
    grid=(N // BN,),
    in_specs=[
        pl.BlockSpec((BN, D), lambda i: (i, 0)),
        pl.BlockSpec((BN, 4), lambda i: (i, 0)),
    ],
    out_specs=[
        pl.BlockSpec((BN, D), lambda i: (i, 0)),
        pl.BlockSpec((BN, D), lambda i: (i, 0)),
    ],
    out_shape=[
        jax.ShapeDtypeStruct((N, D), jnp.float32),
        jax.ShapeDtypeStruct((N, D), jnp.float32),
    ],
)


def _layer1_body(ap_ref, degp_ref, w0_ref, w1_ref, b_ref, hn0_ref, hn1_ref):
    a0 = ap_ref[0] + ap_ref[1]
    a1 = ap_ref[2] + ap_ref[3]
    nd0 = lax.rsqrt(jnp.maximum(degp_ref[:, 1], 1.0))
    nd1 = lax.rsqrt(jnp.maximum(degp_ref[:, 3], 1.0))
    h = (jnp.dot(a0 * nd0[:, None], w0_ref[...],
                 preferred_element_type=jnp.float32)
         + jnp.dot(a1 * nd1[:, None], w1_ref[...],
                   preferred_element_type=jnp.float32)
         + b_ref[...])
    h = jnp.maximum(h, 0.0)
    ns0 = lax.rsqrt(jnp.maximum(degp_ref[:, 0], 1.0))
    ns1 = lax.rsqrt(jnp.maximum(degp_ref[:, 2], 1.0))
    hn0_ref[...] = h * ns0[:, None]
    hn1_ref[...] = h * ns1[:, None]


_layer1 = pl.pallas_call(
    _layer1_body,
    grid=(N // BN,),
    in_specs=[
        pl.BlockSpec((4, BN, D), lambda i: (0, i, 0)),
        pl.BlockSpec((BN, 4), lambda i: (i, 0)),
        pl.BlockSpec((D, D), lambda i: (0, 0)),
        pl.BlockSpec((D, D), lambda i: (0, 0)),
        pl.BlockSpec((1, D), lambda i: (0, 0)),
    ],
    out_specs=[
        pl.BlockSpec((BN, D), lambda i: (i, 0)),
        pl.BlockSpec((BN, D), lambda i: (i, 0)),
    ],
    out_shape=[
        jax.ShapeDtypeStruct((N, D), jnp.float32),
        jax.ShapeDtypeStruct((N, D), jnp.float32),
    ],
)


def _layer2_body(ap_ref, degp_ref, w0_ref, w1_ref, b_ref, out_ref):
    a0 = ap_ref[0] + ap_ref[1]
    a1 = ap_ref[2] + ap_ref[3]
    nd0 = lax.rsqrt(jnp.maximum(degp_ref[:, 1], 1.0))
    nd1 = lax.rsqrt(jnp.maximum(degp_ref[:, 3], 1.0))
    out_ref[...] = (jnp.dot(a0 * nd0[:, None], w0_ref[...],
                            preferred_element_type=jnp.float32)
                    + jnp.dot(a1 * nd1[:, None], w1_ref[...],
                              preferred_element_type=jnp.float32)
                    + b_ref[...])


_layer2 = pl.pallas_call(
    _layer2_body,
    grid=(N // BN,),
    in_specs=[
        pl.BlockSpec((4, BN, D), lambda i: (0, i, 0)),
        pl.BlockSpec((BN, 4), lambda i: (i, 0)),
        pl.BlockSpec((D, D), lambda i: (0, 0)),
        pl.BlockSpec((D, D), lambda i: (0, 0)),
        pl.BlockSpec((1, D), lambda i: (0, 0)),
    ],
    out_specs=pl.BlockSpec((BN, D), lambda i: (i, 0)),
    out_shape=jax.ShapeDtypeStruct((N, D), jnp.float32),
)


def _prep_edges(idx):
    idx = idx.astype(jnp.int32)
    idx = jnp.pad(idx, (0, EPAD - E), constant_values=PADIDX)
    return idx.reshape(NW, NB, BK)


def _pad_table(t):
    return jnp.pad(t, ((0, NPAD - N), (0, 0)))


def kernel(x, edge_index_r0, edge_index_r1, W1_r0, b1_r0, W1_r1, b1_r1,
                 W6_r0, b6_r0, W6_r1, b6_r1):
    s0p = _prep_edges(edge_index_r0[0])
    d0p = _prep_edges(edge_index_r0[1])
    s1p = _prep_edges(edge_index_r1[0])
    d1p = _prep_edges(edge_index_r1[1])

    zerosD = jnp.zeros((RPS, D), jnp.float32)

    idx4 = jnp.stack([s0p.reshape(EPAD), d0p.reshape(EPAD),
                      s1p.reshape(EPAD), d1p.reshape(EPAD)])
    degp = _deg_tc(idx4.reshape(4, 1, EPAD))
    degp8 = degp.reshape(4, NPAD)[:, :N].T

    xn0, xn1 = _prep(x, degp8)
    ap1 = _spmv_kernel(_pad_table(xn0), s0p, d0p,
                       _pad_table(xn1), s1p, d1p, zerosD)
    ap1r = ap1[:, :, :N, :].reshape(4, N, D)

    b1 = (b1_r0 + b1_r1).reshape(1, D)
    hn0, hn1 = _layer1(ap1r, degp8, W1_r0, W1_r1, b1)

    ap2 = _spmv_kernel(_pad_table(hn0), s0p, d0p,
                       _pad_table(hn1), s1p, d1p, zerosD)
    ap2r = ap2[:, :, :N, :].reshape(4, N, D)

    b6 = (b6_r0 + b6_r1).reshape(1, D)
    return _layer2(ap2r, degp8, W6_r0, W6_r1, b6)


# speedup vs baseline: 2.3501x; 1.0722x over previous
import functools

import jax
import jax.numpy as jnp
from jax import lax
from jax.experimental import pallas as pl
from jax.experimental.pallas import tpu as pltpu
from jax.experimental.pallas import tpu_sc as plsc

N = 10000
D = 128
E = 320000
NCORE = 2
NSUB = 16
NW = NCORE * NSUB
BK = 128
NB = 80
EPW = NB * BK
EPAD = NW * EPW
NPAD = 10240
RPS = NPAD // NSUB
PADIDX = N

_mesh = plsc.VectorSubcoreMesh(
    core_axis_name="c", subcore_axis_name="s",
    num_cores=NCORE, num_subcores=NSUB,
)


CH = 16
NCH = NB // CH
HB = NPAD // BK
KD = 2048
NCHK = EPAD // KD


def _deg_body(idx_ref, out_ref):
    g = pl.program_id(1)
    idx = idx_ref[0, 0, :]
    hi = lax.shift_right_logical(idx, 7)
    lo = lax.bitwise_and(idx, 127)
    rh = lax.broadcasted_iota(jnp.int32, (KD, HB), 1)
    rl = lax.broadcasted_iota(jnp.int32, (KD, BK), 1)
    ohh = (hi[:, None] == rh).astype(jnp.bfloat16)
    ohl = (lo[:, None] == rl).astype(jnp.bfloat16)
    part = lax.dot_general(ohh, ohl, (((0,), (0,)), ((), ())),
                           preferred_element_type=jnp.float32)

    @pl.when(g == 0)
    def _():
        out_ref[...] = jnp.zeros_like(out_ref)

    out_ref[...] += part[None]


_deg_tc = pl.pallas_call(
    _deg_body,
    grid=(4, NCHK),
    in_specs=[pl.BlockSpec((1, 1, KD), lambda a, g: (a, 0, g))],
    out_specs=pl.BlockSpec((1, HB, BK), lambda a, g: (a, 0, 0)),
    out_shape=jax.ShapeDtypeStruct((4, HB, BK), jnp.float32),
)


@functools.partial(
    pl.kernel,
    out_type=jax.ShapeDtypeStruct((2, NCORE, NPAD, D), jnp.float32),
    mesh=_mesh,
    scratch_types=[
        pltpu.VMEM((CH, BK), jnp.int32),
        pltpu.VMEM((CH, BK), jnp.int32),
        pltpu.VMEM((BK, D), jnp.float32),
        pltpu.VMEM((BK, D), jnp.float32),
        pltpu.VMEM_SHARED((NPAD, D), jnp.float32),
        pltpu.SemaphoreType.DMA,
        pltpu.SemaphoreType.DMA,
        pltpu.SemaphoreType.DMA,
        pltpu.SemaphoreType.DMA,
    ],
)
def _spmv_kernel(t0, s0, d0, t1, s1, d1, zeros_hbm, out,
                 src_v, dst_v, msg0, msg1, acc, gs0, gs1, ss0, ss1):
    c = lax.axis_index("c")
    s = lax.axis_index("s")
    w = c * NSUB + s
    msgs = (msg0, msg1)
    gsems = (gs0, gs1)
    ssems = (ss0, ss1)
    for rel, (th, sh, dh) in enumerate(((t0, s0, d0), (t1, s1, d1))):
        pltpu.sync_copy(zeros_hbm, acc.at[pl.ds(s * RPS, RPS)])
        plsc.subcore_barrier()

        def chunk(cc, _, th=th, sh=sh, dh=dh):
            pltpu.sync_copy(sh.at[w, pl.ds(cc * CH, CH)], src_v)
            pltpu.sync_copy(dh.at[w, pl.ds(cc * CH, CH)], dst_v)
            gd = [None, None]
            sd = [None, None]
            for j in range(CH):
                b = j & 1
                if sd[b] is not None:
                    sd[b].wait()
                gd[b] = pltpu.async_copy(th.at[src_v.at[j]], msgs[b],
                                         gsems[b])
                if j >= 1:
                    bp = (j - 1) & 1
                    gd[bp].wait()
                    sd[bp] = pltpu.async_copy(msgs[bp],
                                              acc.at[dst_v.at[j - 1]],
                                              ssems[bp], add=True)
            bl = (CH - 1) & 1
            gd[bl].wait()
            sd[bl] = pltpu.async_copy(msgs[bl], acc.at[dst_v.at[CH - 1]],
                                      ssems[bl], add=True)
            sd[0].wait()
            sd[1].wait()
            return 0

        lax.fori_loop(0, NCH, chunk, 0)
        plsc.subcore_barrier()
        pltpu.sync_copy(acc.at[pl.ds(s * RPS, RPS)],
                        out.at[rel, c, pl.ds(s * RPS, RPS)])
        plsc.subcore_barrier()


BN = 1000


def _prep_body(x_ref, degp_ref, xn0_ref, xn1_ref):
    dg0 = degp_ref[:, 0]
    dg1 = degp_ref[:, 2]
    ns0 = lax.rsqrt(jnp.maximum(dg0, 1.0))
    ns1 = lax.rsqrt(jnp.maximum(dg1, 1.0))
    xv = x_ref[...]
    xn0_ref[...] = xv * ns0[:, None]
    xn1_ref[...] = xv * ns1[:, None]


_prep = pl.pallas_call(
    _prep_body,
    grid=(N // BN,),
    in_specs=[
        pl.BlockSpec((BN, D), lambda i: (i, 0)),
        pl.BlockSpec((BN, 4), lambda i: (i, 0)),
    ],
    out_specs=[
        pl.BlockSpec((BN, D), lambda i: (i, 0)),
        pl.BlockSpec((BN, D), lambda i: (i, 0)),
    ],
    out_shape=[
        jax.ShapeDtypeStruct((N, D), jnp.float32),
        jax.ShapeDtypeStruct((N, D), jnp.float32),
    ],
)


def _layer1_body(ap_ref, degp_ref, w0_ref, w1_ref, b_ref, hn0_ref, hn1_ref):
    a0 = ap_ref[0] + ap_ref[1]
    a1 = ap_ref[2] + ap_ref[3]
    nd0 = lax.rsqrt(jnp.maximum(degp_ref[:, 1], 1.0))
    nd1 = lax.rsqrt(jnp.maximum(degp_ref[:, 3], 1.0))
    h = (jnp.dot(a0 * nd0[:, None], w0_ref[...],
                 preferred_element_type=jnp.float32)
         + jnp.dot(a1 * nd1[:, None], w1_ref[...],
                   preferred_element_type=jnp.float32)
         + b_ref[...])
    h = jnp.maximum(h, 0.0)
    ns0 = lax.rsqrt(jnp.maximum(degp_ref[:, 0], 1.0))
    ns1 = lax.rsqrt(jnp.maximum(degp_ref[:, 2], 1.0))
    hn0_ref[...] = h * ns0[:, None]
    hn1_ref[...] = h * ns1[:, None]


_layer1 = pl.pallas_call(
    _layer1_body,
    grid=(N // BN,),
    in_specs=[
        pl.BlockSpec((4, BN, D), lambda i: (0, i, 0)),
        pl.BlockSpec((BN, 4), lambda i: (i, 0)),
        pl.BlockSpec((D, D), lambda i: (0, 0)),
        pl.BlockSpec((D, D), lambda i: (0, 0)),
        pl.BlockSpec((1, D), lambda i: (0, 0)),
    ],
    out_specs=[
        pl.BlockSpec((BN, D), lambda i: (i, 0)),
        pl.BlockSpec((BN, D), lambda i: (i, 0)),
    ],
    out_shape=[
        jax.ShapeDtypeStruct((N, D), jnp.float32),
        jax.ShapeDtypeStruct((N, D), jnp.float32),
    ],
)


def _layer2_body(ap_ref, degp_ref, w0_ref, w1_ref, b_ref, out_ref):
    a0 = ap_ref[0] + ap_ref[1]
    a1 = ap_ref[2] + ap_ref[3]
    nd0 = lax.rsqrt(jnp.maximum(degp_ref[:, 1], 1.0))
    nd1 = lax.rsqrt(jnp.maximum(degp_ref[:, 3], 1.0))
    out_ref[...] = (jnp.dot(a0 * nd0[:, None], w0_ref[...],
                            preferred_element_type=jnp.float32)
                    + jnp.dot(a1 * nd1[:, None], w1_ref[...],
                              preferred_element_type=jnp.float32)
                    + b_ref[...])


_layer2 = pl.pallas_call(
    _layer2_body,
    grid=(N // BN,),
    in_specs=[
        pl.BlockSpec((4, BN, D), lambda i: (0, i, 0)),
        pl.BlockSpec((BN, 4), lambda i: (i, 0)),
        pl.BlockSpec((D, D), lambda i: (0, 0)),
        pl.BlockSpec((D, D), lambda i: (0, 0)),
        pl.BlockSpec((1, D), lambda i: (0, 0)),
    ],
    out_specs=pl.BlockSpec((BN, D), lambda i: (i, 0)),
    out_shape=jax.ShapeDtypeStruct((N, D), jnp.float32),
)


def _prep_edges(idx):
    idx = idx.astype(jnp.int32)
    idx = jnp.pad(idx, (0, EPAD - E), constant_values=PADIDX)
    return idx.reshape(NW, NB, BK)


def _pad_table(t):
    return jnp.pad(t, ((0, NPAD - N), (0, 0)))


def kernel(x, edge_index_r0, edge_index_r1, W1_r0, b1_r0, W1_r1, b1_r1,
                 W6_r0, b6_r0, W6_r1, b6_r1):
    s0p = _prep_edges(edge_index_r0[0])
    d0p = _prep_edges(edge_index_r0[1])
    s1p = _prep_edges(edge_index_r1[0])
    d1p = _prep_edges(edge_index_r1[1])

    zerosD = jnp.zeros((RPS, D), jnp.float32)

    idx4 = jnp.stack([s0p.reshape(EPAD), d0p.reshape(EPAD),
                      s1p.reshape(EPAD), d1p.reshape(EPAD)])
    degp = _deg_tc(idx4.reshape(4, 1, EPAD))
    degp8 = degp.reshape(4, NPAD)[:, :N].T

    xn0, xn1 = _prep(x, degp8)
    ap1 = _spmv_kernel(_pad_table(xn0), s0p, d0p,
                       _pad_table(xn1), s1p, d1p, zerosD)
    ap1r = ap1[:, :, :N, :].reshape(4, N, D)

    b1 = (b1_r0 + b1_r1).reshape(1, D)
    hn0, hn1 = _layer1(ap1r, degp8, W1_r0, W1_r1, b1)

    ap2 = _spmv_kernel(_pad_table(hn0), s0p, d0p,
                       _pad_table(hn1), s1p, d1p, zerosD)
    ap2r = ap2[:, :, :N, :].reshape(4, N, D)

    b6 = (b6_r0 + b6_r1).reshape(1, D)
    return _layer2(ap2r, degp8, W6_r0, W6_r1, b6)
